# baseline (speedup 1.0000x reference)
import functools

import numpy as np
import jax
import jax.numpy as jnp
from jax import lax
from jax.experimental import pallas as pl
from jax.experimental.pallas import tpu as pltpu
from jax.experimental.pallas import tpu_sc as plsc

_CELL = 20.0
_CUTOFF = 2.5
_BOLTZMAN = 0.001987191
_N = 4096
_MAXE = _N * (_N - 1) // 2

_BI = 128
_NTILES = 32
_ROWS = _N // _NTILES
_WPR = _N // 16
_WPT = _ROWS * _WPR
_WCAP = 4096
_ECAP = 8192
_RCAP = _ECAP + 32

_P_NP = (
    np.equal.outer(np.arange(_N) // 16, np.arange(_WPR))
    * (1 << (np.arange(_N) % 16))[:, None]
).astype(np.float32)
_P2_NP = (
    np.equal.outer(np.arange(_WPR) // 16, np.arange(_WPR // 16))
    * (1 << (np.arange(_WPR) % 16))[:, None]
).astype(np.float32)


def _mask_body(q_ref, qt_ref, p_ref, p2_ref, w_ref, sw_ref):
    bi = pl.program_id(0)
    acc = None
    for d in range(3):
        qi = q_ref[:, d : d + 1]
        qj = qt_ref[d : d + 1, :]
        disp = qj - qi
        y = disp * (1.0 / _CELL)
        shift = jnp.where(y > 0.5, _CELL, 0.0) - jnp.where(y < -0.5, _CELL, 0.0)
        disp = disp - shift
        sq = disp * disp
        acc = sq if acc is None else acc + sq
    dist = jnp.sqrt(acc + 1e-12)
    i_ids = bi * _BI + lax.broadcasted_iota(jnp.int32, (_BI, _N), 0)
    j_ids = lax.broadcasted_iota(jnp.int32, (_BI, _N), 1)
    m = (dist < _CUTOFF) & (i_ids < j_ids)
    w = jnp.dot(m.astype(jnp.bfloat16), p_ref[...].astype(jnp.bfloat16),
                preferred_element_type=jnp.float32)
    w_ref[...] = w.astype(jnp.int32)
    sw = jnp.dot((w > 0.0).astype(jnp.bfloat16), p2_ref[...].astype(jnp.bfloat16),
                 preferred_element_type=jnp.float32)
    sw_ref[...] = sw.astype(jnp.int32)


def _mask_words(q, qt, pmat, p2mat):
    return pl.pallas_call(
        _mask_body,
        grid=(_N // _BI,),
        in_specs=[
            pl.BlockSpec((_BI, 3), lambda i: (i, 0)),
            pl.BlockSpec((3, _N), lambda i: (0, 0)),
            pl.BlockSpec((_N, _WPR), lambda i: (0, 0)),
            pl.BlockSpec((_WPR, _WPR // 16), lambda i: (0, 0)),
        ],
        out_specs=(pl.BlockSpec((_BI, _WPR), lambda i: (i, 0)),
                   pl.BlockSpec((_BI, _WPR // 16), lambda i: (i, 0))),
        out_shape=(jax.ShapeDtypeStruct((_N, _WPR), jnp.int32),
                   jax.ShapeDtypeStruct((_N, _WPR // 16), jnp.int32)),
    )(q, qt, pmat, p2mat)


_GCAP = 262144
_GBR = 32
_GC = 1024
_SQRT2 = float(np.sqrt(np.float32(2.0)))
_ULO = float(np.nextafter(np.float32(-1.0), np.float32(0.0)))


def _rotl(x, d):
    return (x << d) | lax.shift_right_logical(x, 32 - d)


def _threefry_bits(k1, k2, cnt):
    ks0, ks1 = k1, k2
    ks2 = k1 ^ k2 ^ jnp.int32(0x1BD11BDA)
    ks = (ks0, ks1, ks2)
    rot = ((13, 15, 26, 6), (17, 29, 16, 24))
    x0 = jnp.zeros_like(cnt) + ks0
    x1 = cnt + ks1
    for i, (ka, kb) in enumerate(((1, 2), (2, 0), (0, 1), (1, 2), (2, 0))):
        for r in rot[i % 2]:
            x0 = x0 + x1
            x1 = _rotl(x1, r)
            x1 = x0 ^ x1
        x0 = x0 + ks[ka]
        x1 = x1 + ks[kb] + jnp.int32(i + 1)
    return x0 ^ x1


def _bits_to_normal(bits):
    fb = lax.shift_right_logical(bits, 9) | jnp.int32(0x3F800000)
    f = lax.bitcast_convert_type(fb, jnp.float32) - 1.0
    lo = jnp.float32(_ULO)
    u = jnp.maximum(lo, f * (1.0 - lo) + lo)
    w = -jnp.log(1.0 - u * u)
    wl = w - 2.5
    p1 = jnp.float32(2.81022636e-08)
    for c in (3.43273939e-07, -3.5233877e-06, -4.39150654e-06, 0.00021858087,
              -0.00125372503, -0.00417768164, 0.246640727, 1.50140941):
        p1 = jnp.float32(c) + p1 * wl
    ws = jnp.sqrt(w) - 3.0
    p2 = jnp.float32(-0.000200214257)
    for c in (0.000100950558, 0.00134934322, -0.00367342844, 0.00573950773,
              -0.0076224613, 0.00943887047, 1.00167406, 2.83297682):
        p2 = jnp.float32(c) + p2 * ws
    p = jnp.where(w < 5.0, p1, p2)
    return _SQRT2 * (p * u)


def _rng_body(key_ref, r1_ref, r2_ref):
    g = pl.program_id(0)
    row = lax.broadcasted_iota(jnp.int32, (_GBR, _GC), 0)
    col = lax.broadcasted_iota(jnp.int32, (_GBR, _GC), 1)
    cnt = (g * _GBR + row) * _GC + col
    r1_ref[...] = _bits_to_normal(_threefry_bits(key_ref[0], key_ref[1], cnt))
    r2_ref[...] = _bits_to_normal(_threefry_bits(key_ref[2], key_ref[3], cnt))


def _rng_streams(keys):
    nrow = _GCAP // _GC
    spec = pl.BlockSpec((_GBR, _GC), lambda i: (i, 0))
    return pl.pallas_call(
        _rng_body,
        grid=(nrow // _GBR,),
        in_specs=[pl.BlockSpec(memory_space=pltpu.SMEM)],
        out_specs=(spec, spec),
        out_shape=(jax.ShapeDtypeStruct((nrow, _GC), jnp.float32),
                   jax.ShapeDtypeStruct((nrow, _GC), jnp.float32)),
    )(keys)


_SC_MESH = plsc.VectorSubcoreMesh(core_axis_name="c", subcore_axis_name="s")


def _take16(x, idx):
    return lax.gather(
        x, idx[:, None],
        lax.GatherDimensionNumbers(
            offset_dims=(), collapsed_slice_dims=(0,), start_index_map=(0,)),
        slice_sizes=(1,),
        mode=lax.GatherScatterMode.PROMISE_IN_BOUNDS)


def _cumsum16(x):
    lanes = lax.iota(jnp.int32, 16)
    cs = x
    for s in (1, 2, 4, 8):
        idx = jnp.maximum(lanes - s, 0)
        sh = _take16(cs, idx)
        cs = cs + jnp.where(lanes >= s, sh, 0)
    return cs


def _popcnt16(x):
    x = x - (lax.shift_right_logical(x, 1) & 0x5555)
    x = (x & 0x3333) + (lax.shift_right_logical(x, 2) & 0x3333)
    x = (x + lax.shift_right_logical(x, 4)) & 0x0F0F
    return (x + lax.shift_right_logical(x, 8)) & 0x1F


_SPT = _WPT // 16


@functools.partial(
    pl.kernel,
    out_type=(
        jax.ShapeDtypeStruct((_NTILES, _ECAP), jnp.int32),
        jax.ShapeDtypeStruct((_NTILES, _ECAP), jnp.int32),
        jax.ShapeDtypeStruct((_NTILES, 16), jnp.int32),
    ),
    mesh=_SC_MESH,
    compiler_params=pltpu.CompilerParams(needs_layout_passes=False),
    scratch_types=[
        pltpu.VMEM((_WPT + 16,), jnp.int32),
        pltpu.VMEM((_SPT + 16,), jnp.int32),
        pltpu.VMEM((_SPT + 16,), jnp.int32),
        pltpu.VMEM((_WCAP + 16,), jnp.int32),
        pltpu.VMEM((_ECAP,), jnp.int32),
        pltpu.VMEM((_ECAP,), jnp.int32),
        pltpu.VMEM((16,), jnp.int32),
    ],
)
def _sc_compact(words_hbm, sup_hbm, ei_hbm, ej_hbm, cnt_hbm,
                wds, sws, sslist, wlist, ei, ej, cstage):
    wid = lax.axis_index("s") * 2 + lax.axis_index("c")
    lanes = lax.iota(jnp.int32, 16)
    pmaskl = (jnp.int32(1) << lanes) - 1
    pltpu.sync_copy(words_hbm.at[pl.ds(wid * _WPT, _WPT)],
                    wds.at[pl.ds(0, _WPT)])
    pltpu.sync_copy(sup_hbm.at[pl.ds(wid * _SPT, _SPT)],
                    sws.at[pl.ds(0, _SPT)])

    def l1(t, c):
        sv = sws[pl.ds(t * 16, 16)]
        m = sv != 0
        cs = _cumsum16(jnp.where(m, 1, 0))
        plsc.store_scatter(sslist, [c + cs - 1], t * 16 + lanes, mask=m)
        return c + cs[15]

    nss = lax.fori_loop(0, _SPT // 16, l1, jnp.int32(0))

    def l2(t, wk):
        s = sslist[pl.ds(t, 16)][0]
        sw = sws[pl.ds(s, 16)][0]
        swv = jnp.full((16,), sw, jnp.int32)
        m = (lax.shift_right_logical(swv, lanes) & 1) != 0
        pos = jnp.minimum(wk + _popcnt16(swv & pmaskl), _WCAP - 1)
        plsc.store_scatter(wlist, [pos], s * 16 + lanes, mask=m)
        return wk + _popcnt16(sw)

    wk = lax.fori_loop(0, nss, l2, jnp.int32(0))

    def l3(t, k):
        wi = wlist[pl.ds(t, 16)][0]
        w = wds[pl.ds(wi, 16)][0]
        wv = jnp.full((16,), w, jnp.int32)
        m = (lax.shift_right_logical(wv, lanes) & 1) != 0
        pos = jnp.minimum(k + _popcnt16(wv & pmaskl), _ECAP - 1)
        i_vec = jnp.full((16,), wid * _ROWS + wi // _WPR, jnp.int32)
        j_vec = (wi % _WPR) * 16 + lanes
        plsc.store_scatter(ei, [pos], i_vec, mask=m)
        plsc.store_scatter(ej, [pos], j_vec, mask=m)
        return k + _popcnt16(w)

    k = lax.fori_loop(0, wk, l3, jnp.int32(0))

    pltpu.sync_copy(ei, ei_hbm.at[wid])
    pltpu.sync_copy(ej, ej_hbm.at[wid])
    cstage[...] = jnp.full((16,), k, jnp.int32)
    pltpu.sync_copy(cstage, cnt_hbm.at[wid])


def _rsqrt(x):
    ix = lax.bitcast_convert_type(x, jnp.int32)
    y = lax.bitcast_convert_type(jnp.int32(0x5F3759DF) - (ix >> 1), jnp.float32)
    for _ in range(4):
        y = y * (1.5 - 0.5 * x * y * y)
    return y


@functools.partial(
    pl.kernel,
    out_type=jax.ShapeDtypeStruct((_NTILES * 3, _N), jnp.float32),
    mesh=_SC_MESH,
    compiler_params=pltpu.CompilerParams(needs_layout_passes=False),
    scratch_types=[
        pltpu.VMEM((_N,), jnp.float32),
        pltpu.VMEM((_N,), jnp.float32),
        pltpu.VMEM((_N,), jnp.float32),
        pltpu.VMEM((_N,), jnp.float32),
        pltpu.VMEM((_N,), jnp.float32),
        pltpu.VMEM((_N,), jnp.float32),
        pltpu.VMEM((_ECAP,), jnp.int32),
        pltpu.VMEM((_ECAP,), jnp.int32),
        pltpu.VMEM((_RCAP,), jnp.float32),
        pltpu.VMEM((_RCAP,), jnp.float32),
        pltpu.VMEM((_N,), jnp.float32),
        pltpu.VMEM((_N,), jnp.float32),
        pltpu.VMEM((_N,), jnp.float32),
        pltpu.VMEM((16,), jnp.int32),
        pltpu.VMEM((16,), jnp.float32),
    ],
)
def _sc_forces(qx_h, qy_h, qz_h, vx_h, vy_h, vz_h, ei_h, ej_h, meta_h, coef_h,
               r1_h, r2_h, out_h,
               qx, qy, qz, vx, vy, vz, ei, ej, r1, r2, ax, ay, az, mst, cst):
    wid = lax.axis_index("s") * 2 + lax.axis_index("c")
    lanes = lax.iota(jnp.int32, 16)
    pltpu.sync_copy(qx_h, qx)
    pltpu.sync_copy(qy_h, qy)
    pltpu.sync_copy(qz_h, qz)
    pltpu.sync_copy(vx_h, vx)
    pltpu.sync_copy(vy_h, vy)
    pltpu.sync_copy(vz_h, vz)
    pltpu.sync_copy(ei_h.at[wid], ei)
    pltpu.sync_copy(ej_h.at[wid], ej)
    pltpu.sync_copy(meta_h.at[wid], mst)
    pltpu.sync_copy(coef_h, cst)
    mv = mst[...]
    cv = cst[...]
    base = mv[0]
    count = mv[1]
    gpar = cv[0]
    gtr = cv[1]
    c1 = cv[2]
    c2 = cv[3]
    base8 = jnp.minimum((base // 8) * 8, _GCAP - _RCAP)
    off = base - base8
    pltpu.sync_copy(r1_h.at[pl.ds(base8, _RCAP)], r1)
    pltpu.sync_copy(r2_h.at[pl.ds(base8, _RCAP)], r2)

    zero = jnp.zeros((16,), jnp.float32)

    def zloop(t, c):
        ax[pl.ds(t * 16, 16)] = zero
        ay[pl.ds(t * 16, 16)] = zero
        az[pl.ds(t * 16, 16)] = zero
        return c

    lax.fori_loop(0, _N // 16, zloop, 0)

    ng = (count + 15) // 16

    def group(g, carry):
        kv = g * 16 + lanes
        valid = kv < count
        iv = jnp.where(valid, ei[pl.ds(g * 16, 16)], 0)
        jv = jnp.where(valid, ej[pl.ds(g * 16, 16)], 0)
        rix = jnp.where(valid, off + kv, 0)
        qxi = plsc.load_gather(qx, [iv])
        qyi = plsc.load_gather(qy, [iv])
        qzi = plsc.load_gather(qz, [iv])
        qxj = plsc.load_gather(qx, [jv])
        qyj = plsc.load_gather(qy, [jv])
        qzj = plsc.load_gather(qz, [jv])
        vxi = plsc.load_gather(vx, [iv])
        vyi = plsc.load_gather(vy, [iv])
        vzi = plsc.load_gather(vz, [iv])
        vxj = plsc.load_gather(vx, [jv])
        vyj = plsc.load_gather(vy, [jv])
        vzj = plsc.load_gather(vz, [jv])
        r1v = plsc.load_gather(r1, [rix])
        r2v = plsc.load_gather(r2, [rix])

        def mimage(d):
            y = d * (1.0 / _CELL)
            return d - (jnp.where(y > 0.5, _CELL, 0.0)
                        - jnp.where(y < -0.5, _CELL, 0.0))

        dx = mimage(qxj - qxi)
        dy = mimage(qyj - qyi)
        dz = mimage(qzj - qzi)
        dd = dx * dx + dy * dy + dz * dz + 1e-12
        rs = _rsqrt(dd)
        ux, uy, uz = dx * rs, dy * rs, dz * rs
        wx, wy, wz = vxi - vxj, vyi - vyj, vzi - vzj
        vpar = wx * ux + wy * uy + wz * uz
        px, py, pz = vpar * ux, vpar * uy, vpar * uz
        tx, ty, tz = wx - px, wy - py, wz - pz
        nt = tx * tx + ty * ty + tz * tz
        rs2 = _rsqrt(nt)
        a1 = r1v * c1
        a2 = r2v * c2 * rs2
        fx = a1 * ux + a2 * tx - px * gpar - tx * gtr
        fy = a1 * uy + a2 * ty - py * gpar - ty * gtr
        fz = a1 * uz + a2 * tz - pz * gpar - tz * gtr
        plsc.addupdate_scatter(ax, [iv], fx, mask=valid)
        plsc.addupdate_scatter(ay, [iv], fy, mask=valid)
        plsc.addupdate_scatter(az, [iv], fz, mask=valid)
        plsc.addupdate_scatter(ax, [jv], -fx, mask=valid)
        plsc.addupdate_scatter(ay, [jv], -fy, mask=valid)
        plsc.addupdate_scatter(az, [jv], -fz, mask=valid)
        return carry

    lax.fori_loop(0, ng, group, 0)

    pltpu.sync_copy(ax, out_h.at[wid * 3 + 0])
    pltpu.sync_copy(ay, out_h.at[wid * 3 + 1])
    pltpu.sync_copy(az, out_h.at[wid * 3 + 2])


def kernel(q, v, T, dt, mass, gamma_parallel, gamma_transverse):
    qt = q.T
    words, sups = _mask_words(q, qt, jnp.asarray(_P_NP), jnp.asarray(_P2_NP))
    ei, ej, cnts = _sc_compact(words.reshape(-1), sups.reshape(-1))
    counts = cnts[:, 0]
    bases = jnp.concatenate(
        [jnp.zeros((1,), jnp.int32), jnp.cumsum(counts)[:-1]])
    meta = jnp.pad(jnp.stack([bases, counts], axis=1), ((0, 0), (0, 14)))

    kd = jax.random.key_data(jax.random.split(jax.random.key(123)))
    keys = lax.bitcast_convert_type(kd, jnp.int32).reshape(4)
    r1, r2 = _rng_streams(keys)
    r1 = r1.reshape(-1)
    r2 = r2.reshape(-1)

    gpar = gamma_parallel[0]
    gtr = gamma_transverse[0]
    c1 = jnp.sqrt(2.0 * gpar / mass[0] * _BOLTZMAN * T * dt) / dt
    c2 = (2 ** 0.5) * jnp.sqrt(2.0 * gtr / mass[0] * _BOLTZMAN * T * dt) / dt
    coef = jnp.stack([gpar, gtr, c1, c2] + [jnp.float32(0)] * 12)

    partials = _sc_forces(
        q[:, 0], q[:, 1], q[:, 2], v[:, 0], v[:, 1], v[:, 2],
        ei, ej, meta, coef, r1, r2)
    return partials.reshape(_NTILES, 3, _N).sum(0).T


# speedup vs baseline: 1.5760x; 1.5760x over previous
import functools

import numpy as np
import jax
import jax.numpy as jnp
from jax import lax
from jax.experimental import pallas as pl
from jax.experimental.pallas import tpu as pltpu
from jax.experimental.pallas import tpu_sc as plsc

_CELL = 20.0
_CUTOFF = 2.5
_BOLTZMAN = 0.001987191
_N = 4096
_MAXE = _N * (_N - 1) // 2

_BI = 128
_NTILES = 32
_ROWS = _N // _NTILES
_WPR = _N // 16
_WPT = _ROWS * _WPR
_WCAP = 4096
_ECAP = 8192
_RCAP = _ECAP + 32

_P_NP = (
    np.equal.outer(np.arange(_N) // 16, np.arange(_WPR))
    * (1 << (np.arange(_N) % 16))[:, None]
).astype(np.float32)
_P2_NP = (
    np.equal.outer(np.arange(_WPR) // 16, np.arange(_WPR // 16))
    * (1 << (np.arange(_WPR) % 16))[:, None]
).astype(np.float32)


def _mask_body(q_ref, qt_ref, p_ref, p2_ref, w_ref, sw_ref):
    bi = pl.program_id(0)
    acc = None
    for d in range(3):
        qi = q_ref[:, d : d + 1]
        qj = qt_ref[d : d + 1, :]
        disp = qj - qi
        y = disp * (1.0 / _CELL)
        shift = jnp.where(y > 0.5, _CELL, 0.0) - jnp.where(y < -0.5, _CELL, 0.0)
        disp = disp - shift
        sq = disp * disp
        acc = sq if acc is None else acc + sq
    dist = jnp.sqrt(acc + 1e-12)
    i_ids = bi * _BI + lax.broadcasted_iota(jnp.int32, (_BI, _N), 0)
    j_ids = lax.broadcasted_iota(jnp.int32, (_BI, _N), 1)
    m = (dist < _CUTOFF) & (i_ids < j_ids)
    w = jnp.dot(m.astype(jnp.bfloat16), p_ref[...].astype(jnp.bfloat16),
                preferred_element_type=jnp.float32)
    w_ref[...] = w.astype(jnp.int32)
    sw = jnp.dot((w > 0.0).astype(jnp.bfloat16), p2_ref[...].astype(jnp.bfloat16),
                 preferred_element_type=jnp.float32)
    sw_ref[...] = sw.astype(jnp.int32)


def _mask_words(q, qt, pmat, p2mat):
    return pl.pallas_call(
        _mask_body,
        grid=(_N // _BI,),
        in_specs=[
            pl.BlockSpec((_BI, 3), lambda i: (i, 0)),
            pl.BlockSpec((3, _N), lambda i: (0, 0)),
            pl.BlockSpec((_N, _WPR), lambda i: (0, 0)),
            pl.BlockSpec((_WPR, _WPR // 16), lambda i: (0, 0)),
        ],
        out_specs=(pl.BlockSpec((_BI, _WPR), lambda i: (i, 0)),
                   pl.BlockSpec((_BI, _WPR // 16), lambda i: (i, 0))),
        out_shape=(jax.ShapeDtypeStruct((_N, _WPR), jnp.int32),
                   jax.ShapeDtypeStruct((_N, _WPR // 16), jnp.int32)),
    )(q, qt, pmat, p2mat)


_GCAP = 262144
_GBR = 32
_GC = 1024
_SQRT2 = float(np.sqrt(np.float32(2.0)))
_ULO = float(np.nextafter(np.float32(-1.0), np.float32(0.0)))


def _rotl(x, d):
    return (x << d) | lax.shift_right_logical(x, 32 - d)


def _threefry_bits(k1, k2, cnt):
    ks0, ks1 = k1, k2
    ks2 = k1 ^ k2 ^ jnp.int32(0x1BD11BDA)
    ks = (ks0, ks1, ks2)
    rot = ((13, 15, 26, 6), (17, 29, 16, 24))
    x0 = jnp.zeros_like(cnt) + ks0
    x1 = cnt + ks1
    for i, (ka, kb) in enumerate(((1, 2), (2, 0), (0, 1), (1, 2), (2, 0))):
        for r in rot[i % 2]:
            x0 = x0 + x1
            x1 = _rotl(x1, r)
            x1 = x0 ^ x1
        x0 = x0 + ks[ka]
        x1 = x1 + ks[kb] + jnp.int32(i + 1)
    return x0 ^ x1


def _bits_to_normal(bits):
    fb = lax.shift_right_logical(bits, 9) | jnp.int32(0x3F800000)
    f = lax.bitcast_convert_type(fb, jnp.float32) - 1.0
    lo = jnp.float32(_ULO)
    u = jnp.maximum(lo, f * (1.0 - lo) + lo)
    w = -jnp.log(1.0 - u * u)
    wl = w - 2.5
    p1 = jnp.float32(2.81022636e-08)
    for c in (3.43273939e-07, -3.5233877e-06, -4.39150654e-06, 0.00021858087,
              -0.00125372503, -0.00417768164, 0.246640727, 1.50140941):
        p1 = jnp.float32(c) + p1 * wl
    ws = jnp.sqrt(w) - 3.0
    p2 = jnp.float32(-0.000200214257)
    for c in (0.000100950558, 0.00134934322, -0.00367342844, 0.00573950773,
              -0.0076224613, 0.00943887047, 1.00167406, 2.83297682):
        p2 = jnp.float32(c) + p2 * ws
    p = jnp.where(w < 5.0, p1, p2)
    return _SQRT2 * (p * u)


def _rng_body(key_ref, r1_ref, r2_ref):
    g = pl.program_id(0)
    row = lax.broadcasted_iota(jnp.int32, (_GBR, _GC), 0)
    col = lax.broadcasted_iota(jnp.int32, (_GBR, _GC), 1)
    cnt = (g * _GBR + row) * _GC + col
    r1_ref[...] = _bits_to_normal(_threefry_bits(key_ref[0], key_ref[1], cnt))
    r2_ref[...] = _bits_to_normal(_threefry_bits(key_ref[2], key_ref[3], cnt))


def _rng_streams(keys):
    nrow = _GCAP // _GC
    spec = pl.BlockSpec((_GBR, _GC), lambda i: (i, 0))
    return pl.pallas_call(
        _rng_body,
        grid=(nrow // _GBR,),
        in_specs=[pl.BlockSpec(memory_space=pltpu.SMEM)],
        out_specs=(spec, spec),
        out_shape=(jax.ShapeDtypeStruct((nrow, _GC), jnp.float32),
                   jax.ShapeDtypeStruct((nrow, _GC), jnp.float32)),
    )(keys)


_SC_MESH = plsc.VectorSubcoreMesh(core_axis_name="c", subcore_axis_name="s")


def _take16(x, idx):
    return lax.gather(
        x, idx[:, None],
        lax.GatherDimensionNumbers(
            offset_dims=(), collapsed_slice_dims=(0,), start_index_map=(0,)),
        slice_sizes=(1,),
        mode=lax.GatherScatterMode.PROMISE_IN_BOUNDS)


def _cumsum16(x):
    lanes = lax.iota(jnp.int32, 16)
    cs = x
    for s in (1, 2, 4, 8):
        idx = jnp.maximum(lanes - s, 0)
        sh = _take16(cs, idx)
        cs = cs + jnp.where(lanes >= s, sh, 0)
    return cs


def _popcnt16(x):
    x = x - (lax.shift_right_logical(x, 1) & 0x5555)
    x = (x & 0x3333) + (lax.shift_right_logical(x, 2) & 0x3333)
    x = (x + lax.shift_right_logical(x, 4)) & 0x0F0F
    return (x + lax.shift_right_logical(x, 8)) & 0x1F


_SPT = _WPT // 16


@functools.partial(
    pl.kernel,
    out_type=(
        jax.ShapeDtypeStruct((_NTILES, _ECAP), jnp.int32),
        jax.ShapeDtypeStruct((_NTILES, _ECAP), jnp.int32),
        jax.ShapeDtypeStruct((_NTILES, 16), jnp.int32),
    ),
    mesh=_SC_MESH,
    compiler_params=pltpu.CompilerParams(needs_layout_passes=False),
    scratch_types=[
        pltpu.VMEM((_WPT + 16,), jnp.int32),
        pltpu.VMEM((_SPT + 16,), jnp.int32),
        pltpu.VMEM((_SPT + 16,), jnp.int32),
        pltpu.VMEM((_WCAP + 16,), jnp.int32),
        pltpu.VMEM((_ECAP,), jnp.int32),
        pltpu.VMEM((_ECAP,), jnp.int32),
        pltpu.VMEM((16,), jnp.int32),
    ],
)
def _sc_compact(words_hbm, sup_hbm, ei_hbm, ej_hbm, cnt_hbm,
                wds, sws, sslist, wlist, ei, ej, cstage):
    wid = lax.axis_index("s") * 2 + lax.axis_index("c")
    lanes = lax.iota(jnp.int32, 16)
    pmaskl = (jnp.int32(1) << lanes) - 1
    zeros16 = jnp.zeros((16,), jnp.int32)
    pltpu.sync_copy(words_hbm.at[pl.ds(wid * _WPT, _WPT)],
                    wds.at[pl.ds(0, _WPT)])
    pltpu.sync_copy(sup_hbm.at[pl.ds(wid * _SPT, _SPT)],
                    sws.at[pl.ds(0, _SPT)])
    wds[pl.ds(_WPT, 16)] = zeros16
    sws[pl.ds(_SPT, 16)] = zeros16

    def l1(t, c):
        sv = sws[pl.ds(t * 16, 16)]
        m = sv != 0
        cs = _cumsum16(jnp.where(m, 1, 0))
        plsc.store_scatter(sslist, [c + cs - 1], t * 16 + lanes, mask=m)
        return c + cs[15]

    nss = lax.fori_loop(0, _SPT // 16, l1, jnp.int32(0))
    plsc.store_scatter(sslist, [nss + lanes], jnp.full((16,), _SPT, jnp.int32))

    def l2(g, wk):
        sis = sslist[pl.ds(g * 16, 16)]
        svs = plsc.load_gather(sws, [sis])
        pcs = _popcnt16(svs)
        cs = _cumsum16(pcs)
        for l in range(16):
            swv = jnp.full((16,), svs[l], jnp.int32)
            m = (lax.shift_right_logical(swv, lanes) & 1) != 0
            b = wk + cs[l] - pcs[l]
            pos = jnp.minimum(b + _popcnt16(swv & pmaskl), _WCAP - 1)
            plsc.store_scatter(wlist, [pos], sis[l] * 16 + lanes, mask=m)
        return wk + cs[15]

    wk = lax.fori_loop(0, (nss + 15) // 16, l2, jnp.int32(0))
    wkc = jnp.minimum(wk, _WCAP)
    plsc.store_scatter(wlist, [wkc + lanes], jnp.full((16,), _WPT, jnp.int32))

    def l3(g, k):
        wis = wlist[pl.ds(g * 16, 16)]
        ws = plsc.load_gather(wds, [wis])
        pcs = _popcnt16(ws)
        cs = _cumsum16(pcs)
        for l in range(16):
            wv = jnp.full((16,), ws[l], jnp.int32)
            m = (lax.shift_right_logical(wv, lanes) & 1) != 0
            b = k + cs[l] - pcs[l]
            pos = jnp.minimum(b + _popcnt16(wv & pmaskl), _ECAP - 1)
            i_vec = jnp.full((16,), wid * _ROWS + wis[l] // _WPR, jnp.int32)
            j_vec = (wis[l] % _WPR) * 16 + lanes
            plsc.store_scatter(ei, [pos], i_vec, mask=m)
            plsc.store_scatter(ej, [pos], j_vec, mask=m)
        return k + cs[15]

    k = lax.fori_loop(0, (wkc + 15) // 16, l3, jnp.int32(0))

    pltpu.sync_copy(ei, ei_hbm.at[wid])
    pltpu.sync_copy(ej, ej_hbm.at[wid])
    cstage[...] = jnp.full((16,), k, jnp.int32)
    pltpu.sync_copy(cstage, cnt_hbm.at[wid])


def _rsqrt(x):
    ix = lax.bitcast_convert_type(x, jnp.int32)
    y = lax.bitcast_convert_type(jnp.int32(0x5F3759DF) - (ix >> 1), jnp.float32)
    for _ in range(4):
        y = y * (1.5 - 0.5 * x * y * y)
    return y


@functools.partial(
    pl.kernel,
    out_type=jax.ShapeDtypeStruct((_NTILES * 3, _N), jnp.float32),
    mesh=_SC_MESH,
    compiler_params=pltpu.CompilerParams(needs_layout_passes=False),
    scratch_types=[
        pltpu.VMEM((_N,), jnp.float32),
        pltpu.VMEM((_N,), jnp.float32),
        pltpu.VMEM((_N,), jnp.float32),
        pltpu.VMEM((_N,), jnp.float32),
        pltpu.VMEM((_N,), jnp.float32),
        pltpu.VMEM((_N,), jnp.float32),
        pltpu.VMEM((_ECAP,), jnp.int32),
        pltpu.VMEM((_ECAP,), jnp.int32),
        pltpu.VMEM((_RCAP,), jnp.float32),
        pltpu.VMEM((_RCAP,), jnp.float32),
        pltpu.VMEM((_N,), jnp.float32),
        pltpu.VMEM((_N,), jnp.float32),
        pltpu.VMEM((_N,), jnp.float32),
        pltpu.VMEM((16,), jnp.int32),
        pltpu.VMEM((16,), jnp.float32),
    ],
)
def _sc_forces(qx_h, qy_h, qz_h, vx_h, vy_h, vz_h, ei_h, ej_h, meta_h, coef_h,
               r1_h, r2_h, out_h,
               qx, qy, qz, vx, vy, vz, ei, ej, r1, r2, ax, ay, az, mst, cst):
    wid = lax.axis_index("s") * 2 + lax.axis_index("c")
    lanes = lax.iota(jnp.int32, 16)
    pltpu.sync_copy(qx_h, qx)
    pltpu.sync_copy(qy_h, qy)
    pltpu.sync_copy(qz_h, qz)
    pltpu.sync_copy(vx_h, vx)
    pltpu.sync_copy(vy_h, vy)
    pltpu.sync_copy(vz_h, vz)
    pltpu.sync_copy(ei_h.at[wid], ei)
    pltpu.sync_copy(ej_h.at[wid], ej)
    pltpu.sync_copy(meta_h.at[wid], mst)
    pltpu.sync_copy(coef_h, cst)
    mv = mst[...]
    cv = cst[...]
    base = mv[0]
    count = mv[1]
    gpar = cv[0]
    gtr = cv[1]
    c1 = cv[2]
    c2 = cv[3]
    base8 = jnp.minimum((base // 8) * 8, _GCAP - _RCAP)
    off = base - base8
    pltpu.sync_copy(r1_h.at[pl.ds(base8, _RCAP)], r1)
    pltpu.sync_copy(r2_h.at[pl.ds(base8, _RCAP)], r2)

    zero = jnp.zeros((16,), jnp.float32)

    def zloop(t, c):
        ax[pl.ds(t * 16, 16)] = zero
        ay[pl.ds(t * 16, 16)] = zero
        az[pl.ds(t * 16, 16)] = zero
        return c

    lax.fori_loop(0, _N // 16, zloop, 0)

    ng = (count + 15) // 16

    def group(g, carry):
        kv = g * 16 + lanes
        valid = kv < count
        iv = jnp.where(valid, ei[pl.ds(g * 16, 16)], 0)
        jv = jnp.where(valid, ej[pl.ds(g * 16, 16)], 0)
        rix = jnp.where(valid, off + kv, 0)
        qxi = plsc.load_gather(qx, [iv])
        qyi = plsc.load_gather(qy, [iv])
        qzi = plsc.load_gather(qz, [iv])
        qxj = plsc.load_gather(qx, [jv])
        qyj = plsc.load_gather(qy, [jv])
        qzj = plsc.load_gather(qz, [jv])
        vxi = plsc.load_gather(vx, [iv])
        vyi = plsc.load_gather(vy, [iv])
        vzi = plsc.load_gather(vz, [iv])
        vxj = plsc.load_gather(vx, [jv])
        vyj = plsc.load_gather(vy, [jv])
        vzj = plsc.load_gather(vz, [jv])
        r1v = plsc.load_gather(r1, [rix])
        r2v = plsc.load_gather(r2, [rix])

        def mimage(d):
            y = d * (1.0 / _CELL)
            return d - (jnp.where(y > 0.5, _CELL, 0.0)
                        - jnp.where(y < -0.5, _CELL, 0.0))

        dx = mimage(qxj - qxi)
        dy = mimage(qyj - qyi)
        dz = mimage(qzj - qzi)
        dd = dx * dx + dy * dy + dz * dz + 1e-12
        rs = _rsqrt(dd)
        ux, uy, uz = dx * rs, dy * rs, dz * rs
        wx, wy, wz = vxi - vxj, vyi - vyj, vzi - vzj
        vpar = wx * ux + wy * uy + wz * uz
        px, py, pz = vpar * ux, vpar * uy, vpar * uz
        tx, ty, tz = wx - px, wy - py, wz - pz
        nt = tx * tx + ty * ty + tz * tz
        rs2 = _rsqrt(nt)
        a1 = r1v * c1
        a2 = r2v * c2 * rs2
        fx = a1 * ux + a2 * tx - px * gpar - tx * gtr
        fy = a1 * uy + a2 * ty - py * gpar - ty * gtr
        fz = a1 * uz + a2 * tz - pz * gpar - tz * gtr
        plsc.addupdate_scatter(ax, [iv], fx, mask=valid)
        plsc.addupdate_scatter(ay, [iv], fy, mask=valid)
        plsc.addupdate_scatter(az, [iv], fz, mask=valid)
        plsc.addupdate_scatter(ax, [jv], -fx, mask=valid)
        plsc.addupdate_scatter(ay, [jv], -fy, mask=valid)
        plsc.addupdate_scatter(az, [jv], -fz, mask=valid)
        return carry

    lax.fori_loop(0, ng, group, 0)

    pltpu.sync_copy(ax, out_h.at[wid * 3 + 0])
    pltpu.sync_copy(ay, out_h.at[wid * 3 + 1])
    pltpu.sync_copy(az, out_h.at[wid * 3 + 2])


def kernel(q, v, T, dt, mass, gamma_parallel, gamma_transverse):
    qt = q.T
    words, sups = _mask_words(q, qt, jnp.asarray(_P_NP), jnp.asarray(_P2_NP))
    ei, ej, cnts = _sc_compact(words.reshape(-1), sups.reshape(-1))
    counts = cnts[:, 0]
    bases = jnp.concatenate(
        [jnp.zeros((1,), jnp.int32), jnp.cumsum(counts)[:-1]])
    meta = jnp.pad(jnp.stack([bases, counts], axis=1), ((0, 0), (0, 14)))

    kd = jax.random.key_data(jax.random.split(jax.random.key(123)))
    keys = lax.bitcast_convert_type(kd, jnp.int32).reshape(4)
    r1, r2 = _rng_streams(keys)
    r1 = r1.reshape(-1)
    r2 = r2.reshape(-1)

    gpar = gamma_parallel[0]
    gtr = gamma_transverse[0]
    c1 = jnp.sqrt(2.0 * gpar / mass[0] * _BOLTZMAN * T * dt) / dt
    c2 = (2 ** 0.5) * jnp.sqrt(2.0 * gtr / mass[0] * _BOLTZMAN * T * dt) / dt
    coef = jnp.stack([gpar, gtr, c1, c2] + [jnp.float32(0)] * 12)

    partials = _sc_forces(
        q[:, 0], q[:, 1], q[:, 2], v[:, 0], v[:, 1], v[:, 2],
        ei, ej, meta, coef, r1, r2)
    return partials.reshape(_NTILES, 3, _N).sum(0).T


# speedup vs baseline: 1.6651x; 1.0565x over previous
import functools

import numpy as np
import jax
import jax.numpy as jnp
from jax import lax
from jax.experimental import pallas as pl
from jax.experimental.pallas import tpu as pltpu
from jax.experimental.pallas import tpu_sc as plsc

_CELL = 20.0
_CUTOFF = 2.5
_BOLTZMAN = 0.001987191
_N = 4096
_MAXE = _N * (_N - 1) // 2

_BI = 256
_JB = 2048
_NTILES = 32
_ROWS = _N // _NTILES
_WPR = _N // 16
_WPT = _ROWS * _WPR
_WCAP = 4096
_ECAP = 8192
_RCAP = _ECAP + 32

_P_NP = (
    np.equal.outer(np.arange(_N) // 16, np.arange(_WPR))
    * (1 << (np.arange(_N) % 16))[:, None]
).astype(np.float32)
_P2_NP = (
    np.equal.outer(np.arange(_WPR) // 16, np.arange(_WPR // 16))
    * (1 << (np.arange(_WPR) % 16))[:, None]
).astype(np.float32)


def _mask_body(q_ref, qt_ref, p_ref, p2_ref, w_ref, sw_ref):
    bi = pl.program_id(0)
    bj = pl.program_id(1)

    @pl.when(bj == 0)
    def _init_sw():
        sw_ref[...] = jnp.zeros((_BI, _WPR // 16), jnp.int32)

    @pl.when((bj + 1) * _JB > bi * _BI)
    def _compute():
        acc = None
        for d in range(3):
            qi = q_ref[:, d : d + 1]
            qj = qt_ref[d : d + 1, :]
            disp = qj - qi
            y = disp * (1.0 / _CELL)
            shift = (jnp.where(y > 0.5, _CELL, 0.0)
                     - jnp.where(y < -0.5, _CELL, 0.0))
            disp = disp - shift
            sq = disp * disp
            acc = sq if acc is None else acc + sq
        dist = jnp.sqrt(acc + 1e-12)
        i_ids = bi * _BI + lax.broadcasted_iota(jnp.int32, (_BI, _JB), 0)
        j_ids = bj * _JB + lax.broadcasted_iota(jnp.int32, (_BI, _JB), 1)
        m = (dist < _CUTOFF) & (i_ids < j_ids)
        w = jnp.dot(m.astype(jnp.bfloat16), p_ref[...].astype(jnp.bfloat16),
                    preferred_element_type=jnp.float32)
        w_ref[...] = w.astype(jnp.int32)
        sw = jnp.dot((w > 0.0).astype(jnp.bfloat16),
                     p2_ref[...].astype(jnp.bfloat16),
                     preferred_element_type=jnp.float32)
        sw_ref[...] = sw_ref[...] + sw.astype(jnp.int32)

    @pl.when((bj + 1) * _JB <= bi * _BI)
    def _skip():
        w_ref[...] = jnp.zeros((_BI, _JB // 16), jnp.int32)


def _mask_words(q, qt, pmat, p2mat):
    return pl.pallas_call(
        _mask_body,
        grid=(_N // _BI, _N // _JB),
        in_specs=[
            pl.BlockSpec((_BI, 3), lambda i, j: (i, 0)),
            pl.BlockSpec((3, _JB), lambda i, j: (0, j)),
            pl.BlockSpec((_JB, _JB // 16), lambda i, j: (j, j)),
            pl.BlockSpec((_JB // 16, _WPR // 16), lambda i, j: (j, 0)),
        ],
        out_specs=(pl.BlockSpec((_BI, _JB // 16), lambda i, j: (i, j)),
                   pl.BlockSpec((_BI, _WPR // 16), lambda i, j: (i, 0))),
        out_shape=(jax.ShapeDtypeStruct((_N, _WPR), jnp.int32),
                   jax.ShapeDtypeStruct((_N, _WPR // 16), jnp.int32)),
    )(q, qt, pmat, p2mat)


_GCAP = 262144
_GBR = 32
_GC = 1024
_SQRT2 = float(np.sqrt(np.float32(2.0)))
_ULO = float(np.nextafter(np.float32(-1.0), np.float32(0.0)))


def _rotl(x, d):
    return (x << d) | lax.shift_right_logical(x, 32 - d)


def _threefry_bits(k1, k2, cnt):
    ks0, ks1 = k1, k2
    ks2 = k1 ^ k2 ^ jnp.int32(0x1BD11BDA)
    ks = (ks0, ks1, ks2)
    rot = ((13, 15, 26, 6), (17, 29, 16, 24))
    x0 = jnp.zeros_like(cnt) + ks0
    x1 = cnt + ks1
    for i, (ka, kb) in enumerate(((1, 2), (2, 0), (0, 1), (1, 2), (2, 0))):
        for r in rot[i % 2]:
            x0 = x0 + x1
            x1 = _rotl(x1, r)
            x1 = x0 ^ x1
        x0 = x0 + ks[ka]
        x1 = x1 + ks[kb] + jnp.int32(i + 1)
    return x0 ^ x1


def _bits_to_normal(bits):
    fb = lax.shift_right_logical(bits, 9) | jnp.int32(0x3F800000)
    f = lax.bitcast_convert_type(fb, jnp.float32) - 1.0
    lo = jnp.float32(_ULO)
    u = jnp.maximum(lo, f * (1.0 - lo) + lo)
    w = -jnp.log(1.0 - u * u)
    wl = w - 2.5
    p1 = jnp.float32(2.81022636e-08)
    for c in (3.43273939e-07, -3.5233877e-06, -4.39150654e-06, 0.00021858087,
              -0.00125372503, -0.00417768164, 0.246640727, 1.50140941):
        p1 = jnp.float32(c) + p1 * wl
    ws = jnp.sqrt(w) - 3.0
    p2 = jnp.float32(-0.000200214257)
    for c in (0.000100950558, 0.00134934322, -0.00367342844, 0.00573950773,
              -0.0076224613, 0.00943887047, 1.00167406, 2.83297682):
        p2 = jnp.float32(c) + p2 * ws
    p = jnp.where(w < 5.0, p1, p2)
    return _SQRT2 * (p * u)


def _rng_body(key_ref, r1_ref, r2_ref):
    g = pl.program_id(0)
    row = lax.broadcasted_iota(jnp.int32, (_GBR, _GC), 0)
    col = lax.broadcasted_iota(jnp.int32, (_GBR, _GC), 1)
    cnt = (g * _GBR + row) * _GC + col
    r1_ref[...] = _bits_to_normal(_threefry_bits(key_ref[0], key_ref[1], cnt))
    r2_ref[...] = _bits_to_normal(_threefry_bits(key_ref[2], key_ref[3], cnt))


def _rng_streams(keys):
    nrow = _GCAP // _GC
    spec = pl.BlockSpec((_GBR, _GC), lambda i: (i, 0))
    return pl.pallas_call(
        _rng_body,
        grid=(nrow // _GBR,),
        in_specs=[pl.BlockSpec(memory_space=pltpu.SMEM)],
        out_specs=(spec, spec),
        out_shape=(jax.ShapeDtypeStruct((nrow, _GC), jnp.float32),
                   jax.ShapeDtypeStruct((nrow, _GC), jnp.float32)),
    )(keys)


_SC_MESH = plsc.VectorSubcoreMesh(core_axis_name="c", subcore_axis_name="s")


def _take16(x, idx):
    return lax.gather(
        x, idx[:, None],
        lax.GatherDimensionNumbers(
            offset_dims=(), collapsed_slice_dims=(0,), start_index_map=(0,)),
        slice_sizes=(1,),
        mode=lax.GatherScatterMode.PROMISE_IN_BOUNDS)


def _cumsum16(x):
    lanes = lax.iota(jnp.int32, 16)
    cs = x
    for s in (1, 2, 4, 8):
        idx = jnp.maximum(lanes - s, 0)
        sh = _take16(cs, idx)
        cs = cs + jnp.where(lanes >= s, sh, 0)
    return cs


def _popcnt16(x):
    x = x - (lax.shift_right_logical(x, 1) & 0x5555)
    x = (x & 0x3333) + (lax.shift_right_logical(x, 2) & 0x3333)
    x = (x + lax.shift_right_logical(x, 4)) & 0x0F0F
    return (x + lax.shift_right_logical(x, 8)) & 0x1F


_SPT = _WPT // 16


@functools.partial(
    pl.kernel,
    out_type=(
        jax.ShapeDtypeStruct((_NTILES, _ECAP), jnp.int32),
        jax.ShapeDtypeStruct((_NTILES, _ECAP), jnp.int32),
        jax.ShapeDtypeStruct((_NTILES, 16), jnp.int32),
    ),
    mesh=_SC_MESH,
    compiler_params=pltpu.CompilerParams(needs_layout_passes=False),
    scratch_types=[
        pltpu.VMEM((_WPT + 16,), jnp.int32),
        pltpu.VMEM((_SPT + 16,), jnp.int32),
        pltpu.VMEM((_SPT + 16,), jnp.int32),
        pltpu.VMEM((_WCAP + 16,), jnp.int32),
        pltpu.VMEM((_ECAP,), jnp.int32),
        pltpu.VMEM((_ECAP,), jnp.int32),
        pltpu.VMEM((16,), jnp.int32),
    ],
)
def _sc_compact(words_hbm, sup_hbm, ei_hbm, ej_hbm, cnt_hbm,
                wds, sws, sslist, wlist, ei, ej, cstage):
    wid = lax.axis_index("s") * 2 + lax.axis_index("c")
    lanes = lax.iota(jnp.int32, 16)
    pmaskl = (jnp.int32(1) << lanes) - 1
    zeros16 = jnp.zeros((16,), jnp.int32)
    pltpu.sync_copy(words_hbm.at[pl.ds(wid * _WPT, _WPT)],
                    wds.at[pl.ds(0, _WPT)])
    pltpu.sync_copy(sup_hbm.at[pl.ds(wid * _SPT, _SPT)],
                    sws.at[pl.ds(0, _SPT)])
    wds[pl.ds(_WPT, 16)] = zeros16
    sws[pl.ds(_SPT, 16)] = zeros16

    def l1(t, c):
        sv = sws[pl.ds(t * 16, 16)]
        m = sv != 0
        cs = _cumsum16(jnp.where(m, 1, 0))
        plsc.store_scatter(sslist, [c + cs - 1], t * 16 + lanes, mask=m)
        return c + cs[15]

    nss = lax.fori_loop(0, _SPT // 16, l1, jnp.int32(0))
    plsc.store_scatter(sslist, [nss + lanes], jnp.full((16,), _SPT, jnp.int32))

    def l2(g, wk):
        sis = sslist[pl.ds(g * 16, 16)]
        svs = plsc.load_gather(sws, [sis])
        pcs = _popcnt16(svs)
        cs = _cumsum16(pcs)
        for l in range(16):
            swv = jnp.full((16,), svs[l], jnp.int32)
            m = (lax.shift_right_logical(swv, lanes) & 1) != 0
            b = wk + cs[l] - pcs[l]
            pos = jnp.minimum(b + _popcnt16(swv & pmaskl), _WCAP - 1)
            plsc.store_scatter(wlist, [pos], sis[l] * 16 + lanes, mask=m)
        return wk + cs[15]

    wk = lax.fori_loop(0, (nss + 15) // 16, l2, jnp.int32(0))
    wkc = jnp.minimum(wk, _WCAP)
    plsc.store_scatter(wlist, [wkc + lanes], jnp.full((16,), _WPT, jnp.int32))

    def l3(g, k):
        wis = wlist[pl.ds(g * 16, 16)]
        ws = plsc.load_gather(wds, [wis])
        pcs = _popcnt16(ws)
        cs = _cumsum16(pcs)
        for l in range(16):
            wv = jnp.full((16,), ws[l], jnp.int32)
            m = (lax.shift_right_logical(wv, lanes) & 1) != 0
            b = k + cs[l] - pcs[l]
            pos = jnp.minimum(b + _popcnt16(wv & pmaskl), _ECAP - 1)
            i_vec = jnp.full((16,), wid * _ROWS + wis[l] // _WPR, jnp.int32)
            j_vec = (wis[l] % _WPR) * 16 + lanes
            plsc.store_scatter(ei, [pos], i_vec, mask=m)
            plsc.store_scatter(ej, [pos], j_vec, mask=m)
        return k + cs[15]

    k = lax.fori_loop(0, (wkc + 15) // 16, l3, jnp.int32(0))

    pltpu.sync_copy(ei, ei_hbm.at[wid])
    pltpu.sync_copy(ej, ej_hbm.at[wid])
    cstage[...] = jnp.full((16,), k, jnp.int32)
    pltpu.sync_copy(cstage, cnt_hbm.at[wid])


def _rsqrt(x):
    ix = lax.bitcast_convert_type(x, jnp.int32)
    y = lax.bitcast_convert_type(jnp.int32(0x5F3759DF) - (ix >> 1), jnp.float32)
    for _ in range(4):
        y = y * (1.5 - 0.5 * x * y * y)
    return y


@functools.partial(
    pl.kernel,
    out_type=jax.ShapeDtypeStruct((_NTILES * 3, _N), jnp.float32),
    mesh=_SC_MESH,
    compiler_params=pltpu.CompilerParams(needs_layout_passes=False),
    scratch_types=[
        pltpu.VMEM((_N,), jnp.float32),
        pltpu.VMEM((_N,), jnp.float32),
        pltpu.VMEM((_N,), jnp.float32),
        pltpu.VMEM((_N,), jnp.float32),
        pltpu.VMEM((_N,), jnp.float32),
        pltpu.VMEM((_N,), jnp.float32),
        pltpu.VMEM((_ECAP,), jnp.int32),
        pltpu.VMEM((_ECAP,), jnp.int32),
        pltpu.VMEM((_RCAP,), jnp.float32),
        pltpu.VMEM((_RCAP,), jnp.float32),
        pltpu.VMEM((_N,), jnp.float32),
        pltpu.VMEM((_N,), jnp.float32),
        pltpu.VMEM((_N,), jnp.float32),
        pltpu.VMEM((16,), jnp.int32),
        pltpu.VMEM((16,), jnp.float32),
    ],
)
def _sc_forces(qx_h, qy_h, qz_h, vx_h, vy_h, vz_h, ei_h, ej_h, meta_h, coef_h,
               r1_h, r2_h, out_h,
               qx, qy, qz, vx, vy, vz, ei, ej, r1, r2, ax, ay, az, mst, cst):
    wid = lax.axis_index("s") * 2 + lax.axis_index("c")
    lanes = lax.iota(jnp.int32, 16)
    pltpu.sync_copy(qx_h, qx)
    pltpu.sync_copy(qy_h, qy)
    pltpu.sync_copy(qz_h, qz)
    pltpu.sync_copy(vx_h, vx)
    pltpu.sync_copy(vy_h, vy)
    pltpu.sync_copy(vz_h, vz)
    pltpu.sync_copy(ei_h.at[wid], ei)
    pltpu.sync_copy(ej_h.at[wid], ej)
    pltpu.sync_copy(meta_h.at[wid], mst)
    pltpu.sync_copy(coef_h, cst)
    mv = mst[...]
    cv = cst[...]
    base = mv[0]
    count = mv[1]
    gpar = cv[0]
    gtr = cv[1]
    c1 = cv[2]
    c2 = cv[3]
    base8 = jnp.minimum((base // 8) * 8, _GCAP - _RCAP)
    off = base - base8
    pltpu.sync_copy(r1_h.at[pl.ds(base8, _RCAP)], r1)
    pltpu.sync_copy(r2_h.at[pl.ds(base8, _RCAP)], r2)

    zero = jnp.zeros((16,), jnp.float32)

    def zloop(t, c):
        ax[pl.ds(t * 16, 16)] = zero
        ay[pl.ds(t * 16, 16)] = zero
        az[pl.ds(t * 16, 16)] = zero
        return c

    lax.fori_loop(0, _N // 16, zloop, 0)

    ng = (count + 15) // 16

    def group(g, carry):
        kv = g * 16 + lanes
        valid = kv < count
        iv = jnp.where(valid, ei[pl.ds(g * 16, 16)], 0)
        jv = jnp.where(valid, ej[pl.ds(g * 16, 16)], 0)
        rix = jnp.where(valid, off + kv, 0)
        qxi = plsc.load_gather(qx, [iv])
        qyi = plsc.load_gather(qy, [iv])
        qzi = plsc.load_gather(qz, [iv])
        qxj = plsc.load_gather(qx, [jv])
        qyj = plsc.load_gather(qy, [jv])
        qzj = plsc.load_gather(qz, [jv])
        vxi = plsc.load_gather(vx, [iv])
        vyi = plsc.load_gather(vy, [iv])
        vzi = plsc.load_gather(vz, [iv])
        vxj = plsc.load_gather(vx, [jv])
        vyj = plsc.load_gather(vy, [jv])
        vzj = plsc.load_gather(vz, [jv])
        r1v = plsc.load_gather(r1, [rix])
        r2v = plsc.load_gather(r2, [rix])

        def mimage(d):
            y = d * (1.0 / _CELL)
            return d - (jnp.where(y > 0.5, _CELL, 0.0)
                        - jnp.where(y < -0.5, _CELL, 0.0))

        dx = mimage(qxj - qxi)
        dy = mimage(qyj - qyi)
        dz = mimage(qzj - qzi)
        dd = dx * dx + dy * dy + dz * dz + 1e-12
        rs = _rsqrt(dd)
        ux, uy, uz = dx * rs, dy * rs, dz * rs
        wx, wy, wz = vxi - vxj, vyi - vyj, vzi - vzj
        vpar = wx * ux + wy * uy + wz * uz
        px, py, pz = vpar * ux, vpar * uy, vpar * uz
        tx, ty, tz = wx - px, wy - py, wz - pz
        nt = tx * tx + ty * ty + tz * tz
        rs2 = _rsqrt(nt)
        a1 = r1v * c1
        a2 = r2v * c2 * rs2
        fx = a1 * ux + a2 * tx - px * gpar - tx * gtr
        fy = a1 * uy + a2 * ty - py * gpar - ty * gtr
        fz = a1 * uz + a2 * tz - pz * gpar - tz * gtr
        plsc.addupdate_scatter(ax, [iv], fx, mask=valid)
        plsc.addupdate_scatter(ay, [iv], fy, mask=valid)
        plsc.addupdate_scatter(az, [iv], fz, mask=valid)
        plsc.addupdate_scatter(ax, [jv], -fx, mask=valid)
        plsc.addupdate_scatter(ay, [jv], -fy, mask=valid)
        plsc.addupdate_scatter(az, [jv], -fz, mask=valid)
        return carry

    lax.fori_loop(0, ng, group, 0)

    pltpu.sync_copy(ax, out_h.at[wid * 3 + 0])
    pltpu.sync_copy(ay, out_h.at[wid * 3 + 1])
    pltpu.sync_copy(az, out_h.at[wid * 3 + 2])


def kernel(q, v, T, dt, mass, gamma_parallel, gamma_transverse):
    qt = q.T
    words, sups = _mask_words(q, qt, jnp.asarray(_P_NP), jnp.asarray(_P2_NP))
    ei, ej, cnts = _sc_compact(words.reshape(-1), sups.reshape(-1))
    counts = cnts[:, 0]
    bases = jnp.concatenate(
        [jnp.zeros((1,), jnp.int32), jnp.cumsum(counts)[:-1]])
    meta = jnp.pad(jnp.stack([bases, counts], axis=1), ((0, 0), (0, 14)))

    kd = jax.random.key_data(jax.random.split(jax.random.key(123)))
    keys = lax.bitcast_convert_type(kd, jnp.int32).reshape(4)
    r1, r2 = _rng_streams(keys)
    r1 = r1.reshape(-1)
    r2 = r2.reshape(-1)

    gpar = gamma_parallel[0]
    gtr = gamma_transverse[0]
    c1 = jnp.sqrt(2.0 * gpar / mass[0] * _BOLTZMAN * T * dt) / dt
    c2 = (2 ** 0.5) * jnp.sqrt(2.0 * gtr / mass[0] * _BOLTZMAN * T * dt) / dt
    coef = jnp.stack([gpar, gtr, c1, c2] + [jnp.float32(0)] * 12)

    partials = _sc_forces(
        q[:, 0], q[:, 1], q[:, 2], v[:, 0], v[:, 1], v[:, 2],
        ei, ej, meta, coef, r1, r2)
    return partials.reshape(_NTILES, 3, _N).sum(0).T
